# baseline (speedup 1.0000x reference)
import functools

import jax
import jax.numpy as jnp
from jax import lax
from jax.experimental import pallas as pl
from jax.experimental.pallas import tpu as pltpu
from jax.experimental.pallas import tpu_sc as plsc

N = 38400
E = 614400
F = 128
H = 16
NPG = 120
G = N // NPG
RO_H = 64
RO_OUT = 960

NC, NS = 2, 16
NW = NC * NS
SUB = 128
CH = 1280
EPW = E // NW
EPT = E // NS
SCH = 640
NPT = N // NS

def _sc_mesh():
    return plsc.VectorSubcoreMesh(core_axis_name="c", subcore_axis_name="s")


def _pre_body(x_ref, w1_ref, b1_ref, w2_ref, b2_ref, p_ref, m_ref):
    x = x_ref[...].astype(jnp.float32)
    p = jnp.dot(x, w1_ref[...], preferred_element_type=jnp.float32)
    p_ref[...] = p
    z = jnp.maximum(2.0 * p + b1_ref[...], 0.0)
    m = jnp.dot(z, w2_ref[...], preferred_element_type=jnp.float32) + b2_ref[...]
    m_ref[...] = jnp.maximum(m, 0.0).astype(jnp.bfloat16)


def _pre(x, w1, b1, w2, b2):
    R = 4800
    return pl.pallas_call(
        _pre_body,
        grid=(N // R,),
        in_specs=[
            pl.BlockSpec((R, F), lambda i: (i, 0)),
            pl.BlockSpec((F, H), lambda i: (0, 0)),
            pl.BlockSpec((1, H), lambda i: (0, 0)),
            pl.BlockSpec((H, F), lambda i: (0, 0)),
            pl.BlockSpec((1, F), lambda i: (0, 0)),
        ],
        out_specs=[
            pl.BlockSpec((R, H), lambda i: (i, 0)),
            pl.BlockSpec((R, F), lambda i: (i, 0)),
        ],
        out_shape=[
            jax.ShapeDtypeStruct((N, H), jnp.float32),
            jax.ShapeDtypeStruct((N, F), jnp.bfloat16),
        ],
    )(x, w1, b1, w2, b2)


def _gather_body(p_hbm, src_hbm, dst_hbm, z_hbm, si, di, rz,
                 sem_i, sem_g, sem_w):
    wid = lax.axis_index("s") * NC + lax.axis_index("c")
    base = wid * EPW
    nit = EPW // CH
    nsub = CH // SUB

    def ids_slab(i):
        return pl.ds((base + i * CH) // SUB, nsub)

    def fire_ids(i, b):
        pltpu.async_copy(src_hbm.at[ids_slab(i), :], si.at[b], sem_i)
        pltpu.async_copy(dst_hbm.at[ids_slab(i), :], di.at[b], sem_i)

    def drain_ids(i, b):
        pltpu.make_async_copy(src_hbm.at[ids_slab(i), :], si.at[b], sem_i).wait()
        pltpu.make_async_copy(dst_hbm.at[ids_slab(i), :], di.at[b], sem_i).wait()

    def fire_rows(i, b, idx, add):
        for j in range(nsub):
            pltpu.async_copy(p_hbm.at[idx.at[b, j]],
                             rz.at[b, pl.ds(j * SUB, SUB), :], sem_g, add=add)

    def drain_rows(i, b, idx):
        for j in range(nsub):
            pltpu.make_async_copy(p_hbm.at[idx.at[b, j]],
                                  rz.at[b, pl.ds(j * SUB, SUB), :], sem_g).wait()

    def fire_wb(i, b):
        pltpu.async_copy(rz.at[b], z_hbm.at[pl.ds(base + i * CH, CH), :], sem_w)

    def drain_wb(i, b):
        pltpu.make_async_copy(rz.at[b], z_hbm.at[pl.ds(base + i * CH, CH), :],
                              sem_w).wait()

    fire_ids(0, 0)

    def body(i, carry):
        b = lax.rem(i, 2)
        drain_ids(i, b)
        fire_rows(i, b, si, False)

        @pl.when(i + 1 < nit)
        def _():
            fire_ids(i + 1, 1 - b)

        drain_rows(i, b, si)
        fire_rows(i, b, di, True)

        @pl.when(i >= 1)
        def _():
            drain_wb(i - 1, 1 - b)

        drain_rows(i, b, di)
        fire_wb(i, b)
        return carry

    lax.fori_loop(0, nit, body, 0)
    drain_wb(nit - 1, lax.rem(nit - 1, 2))


def _gather(p, src2, dst2):
    kernel_fn = functools.partial(
        pl.kernel,
        out_type=jax.ShapeDtypeStruct((E, H), jnp.float32),
        mesh=_sc_mesh(),
        scratch_types=[
            pltpu.VMEM((2, CH // SUB, SUB), jnp.int32),
            pltpu.VMEM((2, CH // SUB, SUB), jnp.int32),
            pltpu.VMEM((2, CH, H), jnp.float32),
            pltpu.SemaphoreType.DMA,
            pltpu.SemaphoreType.DMA,
            pltpu.SemaphoreType.DMA,
        ],
        compiler_params=pltpu.CompilerParams(use_tc_tiling_on_sc=False),
    )(_gather_body)
    return kernel_fn(p, src2, dst2)


def _msg_body(z_ref, b1_ref, w2_ref, b2_ref, m_ref):
    z = jnp.maximum(z_ref[...] + b1_ref[...], 0.0)
    m = jnp.dot(z, w2_ref[...], preferred_element_type=jnp.float32) + b2_ref[...]
    m_ref[...] = jnp.maximum(m, 0.0).astype(jnp.bfloat16)


def _msg(z, b1, w2, b2):
    R = 4096
    return pl.pallas_call(
        _msg_body,
        grid=(E // R,),
        in_specs=[
            pl.BlockSpec((R, H), lambda i: (i, 0)),
            pl.BlockSpec((1, H), lambda i: (0, 0)),
            pl.BlockSpec((H, F), lambda i: (0, 0)),
            pl.BlockSpec((1, F), lambda i: (0, 0)),
        ],
        out_specs=pl.BlockSpec((R, F), lambda i: (i, 0)),
        out_shape=jax.ShapeDtypeStruct((E, F), jnp.bfloat16),
    )(z, b1, w2, b2)


def _scatter_body(m_hbm, dst_hbm, mself_hbm, out_hbm, acc, di, rows,
                  sem_l, sem_a):
    c = lax.axis_index("c")
    s = lax.axis_index("s")

    nit = EPT // SCH
    nsub = SCH // SUB

    for gl in range(2):
        g = 2 * c + gl
        col = g * 32
        pltpu.sync_copy(
            mself_hbm.at[pl.ds(s * NPT, NPT), pl.ds(col, 32)],
            acc.at[pl.ds(s * NPT, NPT), :])
        plsc.subcore_barrier()

        def fire_load(i, b):
            off = s * EPT + i * SCH
            pltpu.async_copy(dst_hbm.at[pl.ds(off // SUB, nsub), :],
                             di.at[b], sem_l)
            pltpu.async_copy(m_hbm.at[pl.ds(off, SCH), pl.ds(col, 32)],
                             rows.at[b], sem_l)

        def drain_load(i, b):
            off = s * EPT + i * SCH
            pltpu.make_async_copy(dst_hbm.at[pl.ds(off // SUB, nsub), :],
                                  di.at[b], sem_l).wait()
            pltpu.make_async_copy(m_hbm.at[pl.ds(off, SCH), pl.ds(col, 32)],
                                  rows.at[b], sem_l).wait()

        def fire_adds(b):
            for j in range(nsub):
                pltpu.async_copy(rows.at[b, pl.ds(j * SUB, SUB), :],
                                 acc.at[di.at[b, j]], sem_a, add=True)

        def drain_adds(b):
            for j in range(nsub):
                pltpu.make_async_copy(rows.at[b, pl.ds(j * SUB, SUB), :],
                                      acc.at[di.at[b, j]], sem_a).wait()

        fire_load(0, 0)

        def body(i, carry):
            b = lax.rem(i, 2)
            drain_load(i, b)

            @pl.when(i >= 1)
            def _():
                drain_adds(1 - b)

            @pl.when(i + 1 < nit)
            def _():
                fire_load(i + 1, 1 - b)

            fire_adds(b)
            return carry

        lax.fori_loop(0, nit, body, 0)
        drain_adds(lax.rem(nit - 1, 2))
        plsc.subcore_barrier()
        pltpu.sync_copy(
            acc.at[pl.ds(s * NPT, NPT), :],
            out_hbm.at[pl.ds(s * NPT, NPT), pl.ds(col, 32)])
        plsc.subcore_barrier()


def _scatter(m, dst2, mself):
    kernel_fn = functools.partial(
        pl.kernel,
        out_type=jax.ShapeDtypeStruct((N, F), jnp.bfloat16),
        mesh=_sc_mesh(),
        scratch_types=[
            pltpu.VMEM_SHARED((N, 32), jnp.bfloat16),
            pltpu.VMEM((2, SCH // SUB, SUB), jnp.int32),
            pltpu.VMEM((2, SCH, 32), jnp.bfloat16),
            pltpu.SemaphoreType.DMA,
            pltpu.SemaphoreType.DMA,
        ],
        compiler_params=pltpu.CompilerParams(use_tc_tiling_on_sc=False),
    )(_scatter_body)
    return kernel_fn(m, dst2, mself)


def _readout_body(y_ref, wr1_ref, br1_ref, wr2_ref, br2_ref, out_ref, acc):
    k = pl.program_id(0)

    @pl.when(k == 0)
    def _():
        acc[...] = jnp.zeros_like(acc)

    acc[...] += jnp.dot(y_ref[...].astype(jnp.float32), wr1_ref[...],
                        preferred_element_type=jnp.float32)

    @pl.when(k == pl.num_programs(0) - 1)
    def _():
        h = jnp.maximum(acc[...] + br1_ref[...], 0.0)
        o = jnp.dot(h, wr2_ref[...], preferred_element_type=jnp.float32)
        out_ref[...] = jnp.maximum(o + br2_ref[...], 0.0)


def _readout(y, wr1, br1, wr2, br2):
    K = NPG * F
    KB = 3840
    return pl.pallas_call(
        _readout_body,
        grid=(K // KB,),
        in_specs=[
            pl.BlockSpec((G, KB), lambda i: (0, i)),
            pl.BlockSpec((KB, RO_H), lambda i: (i, 0)),
            pl.BlockSpec((1, RO_H), lambda i: (0, 0)),
            pl.BlockSpec((RO_H, RO_OUT), lambda i: (0, 0)),
            pl.BlockSpec((1, RO_OUT), lambda i: (0, 0)),
        ],
        out_specs=pl.BlockSpec((G, RO_OUT), lambda i: (0, 0)),
        out_shape=jax.ShapeDtypeStruct((G, RO_OUT), jnp.float32),
        scratch_shapes=[pltpu.VMEM((G, RO_H), jnp.float32)],
    )(y, wr1, br1, wr2, br2)


def _layer(h, src2, dst2, w1, b1, w2, b2):
    p, mself = _pre(h, w1, b1, w2, b2)
    z = _gather(p, src2, dst2)
    m = _msg(z, b1, w2, b2)
    return _scatter(m, dst2, mself)


def kernel(x, edge_index, W1a, b1a, W2a, b2a, W1b, b1b, W2b, b2b,
           Wr1, br1, Wr2, br2):
    src2 = edge_index[0].reshape(E // SUB, SUB)
    dst2 = edge_index[1].reshape(E // SUB, SUB)
    b1a2 = b1a.reshape(1, H)
    b2a2 = b2a.reshape(1, F)
    b1b2 = b1b.reshape(1, H)
    b2b2 = b2b.reshape(1, F)

    x1 = _layer(x, src2, dst2, W1a, b1a2, W2a, b2a2)
    x2 = _layer(x1, src2, dst2, W1b, b1b2, W2b, b2b2)

    y = x2.reshape(G, NPG * F)
    return _readout(y, Wr1, br1.reshape(1, RO_H), Wr2, br2.reshape(1, RO_OUT))


# speedup vs baseline: 1.8047x; 1.8047x over previous
import functools

import jax
import jax.numpy as jnp
from jax import lax
from jax.experimental import pallas as pl
from jax.experimental.pallas import tpu as pltpu
from jax.experimental.pallas import tpu_sc as plsc

N = 38400
E = 614400
F = 128
H = 16
NPG = 120
G = N // NPG
RO_H = 64
RO_OUT = 960

NC, NS = 2, 16
NW = NC * NS
SUB = 128
CH = 1920
EPW = E // NW
EPT = E // NS
SCH = 640
NPT = N // NS

def _sc_mesh():
    return plsc.VectorSubcoreMesh(core_axis_name="c", subcore_axis_name="s")


def _pre_body(x_ref, w1_ref, b1_ref, w2_ref, b2_ref, p_ref, m_ref):
    p = jnp.dot(x_ref[...], w1_ref[...], preferred_element_type=jnp.float32)
    p_ref[...] = p
    z = jnp.maximum(2.0 * p + b1_ref[...], 0.0)
    m = jnp.dot(z, w2_ref[...], preferred_element_type=jnp.float32) + b2_ref[...]
    m_ref[...] = jnp.maximum(m, 0.0)


def _pre(x, w1, b1, w2, b2):
    R = 4800
    return pl.pallas_call(
        _pre_body,
        grid=(N // R,),
        in_specs=[
            pl.BlockSpec((R, F), lambda i: (i, 0)),
            pl.BlockSpec((F, H), lambda i: (0, 0)),
            pl.BlockSpec((1, H), lambda i: (0, 0)),
            pl.BlockSpec((H, F), lambda i: (0, 0)),
            pl.BlockSpec((1, F), lambda i: (0, 0)),
        ],
        out_specs=[
            pl.BlockSpec((R, H), lambda i: (i, 0)),
            pl.BlockSpec((R, F), lambda i: (i, 0)),
        ],
        out_shape=[
            jax.ShapeDtypeStruct((N, H), jnp.float32),
            jax.ShapeDtypeStruct((N, F), jnp.float32),
        ],
    )(x, w1, b1, w2, b2)


def _gather_body(p_hbm, src_hbm, dst_hbm, z_hbm, si, di, rz,
                 sem_i, sem_g, sem_w, *, e0, ne):
    wid = lax.axis_index("s") * NC + lax.axis_index("c")
    epw = ne // NW
    base = e0 + wid * epw
    nit = epw // CH
    nsub = CH // SUB

    def ids_slab(i):
        return pl.ds((base + i * CH) // SUB, nsub)

    def fire_ids(i, b):
        pltpu.async_copy(src_hbm.at[ids_slab(i), :], si.at[b], sem_i)
        pltpu.async_copy(dst_hbm.at[ids_slab(i), :], di.at[b], sem_i)

    def drain_ids(i, b):
        pltpu.make_async_copy(src_hbm.at[ids_slab(i), :], si.at[b], sem_i).wait()
        pltpu.make_async_copy(dst_hbm.at[ids_slab(i), :], di.at[b], sem_i).wait()

    def fire_rows(i, b, idx, add):
        for j in range(nsub):
            pltpu.async_copy(p_hbm.at[idx.at[b, j]],
                             rz.at[b, pl.ds(j * SUB, SUB), :], sem_g, add=add)

    def drain_rows(i, b, idx):
        for j in range(nsub):
            pltpu.make_async_copy(p_hbm.at[idx.at[b, j]],
                                  rz.at[b, pl.ds(j * SUB, SUB), :], sem_g).wait()

    def fire_wb(i, b):
        pltpu.async_copy(rz.at[b], z_hbm.at[pl.ds(base - e0 + i * CH, CH), :],
                         sem_w)

    def drain_wb(i, b):
        pltpu.make_async_copy(rz.at[b],
                              z_hbm.at[pl.ds(base - e0 + i * CH, CH), :],
                              sem_w).wait()

    fire_ids(0, 0)

    def body(i, carry):
        b = lax.rem(i, 2)
        drain_ids(i, b)
        fire_rows(i, b, si, False)

        @pl.when(i + 1 < nit)
        def _():
            fire_ids(i + 1, 1 - b)

        drain_rows(i, b, si)
        fire_rows(i, b, di, True)

        @pl.when(i >= 1)
        def _():
            drain_wb(i - 1, 1 - b)

        drain_rows(i, b, di)
        fire_wb(i, b)
        return carry

    lax.fori_loop(0, nit, body, 0)
    drain_wb(nit - 1, lax.rem(nit - 1, 2))


def _gather(p, src2, dst2, e0, ne):
    kernel_fn = functools.partial(
        pl.kernel,
        out_type=jax.ShapeDtypeStruct((ne, H), jnp.float32),
        mesh=_sc_mesh(),
        scratch_types=[
            pltpu.VMEM((2, CH // SUB, SUB), jnp.int32),
            pltpu.VMEM((2, CH // SUB, SUB), jnp.int32),
            pltpu.VMEM((2, CH, H), jnp.float32),
            pltpu.SemaphoreType.DMA,
            pltpu.SemaphoreType.DMA,
            pltpu.SemaphoreType.DMA,
        ],
        compiler_params=pltpu.CompilerParams(use_tc_tiling_on_sc=False),
    )(functools.partial(_gather_body, e0=e0, ne=ne))
    return kernel_fn(p, src2, dst2)


def _msg_body(z_ref, b1_ref, w2_ref, b2_ref, m_ref):
    z = jnp.maximum(z_ref[...] + b1_ref[...], 0.0)
    m = jnp.dot(z, w2_ref[...], preferred_element_type=jnp.float32) + b2_ref[...]
    m_ref[...] = jnp.maximum(m, 0.0)


def _msg(z, b1, w2, b2):
    R = 4096
    ne = z.shape[0]
    return pl.pallas_call(
        _msg_body,
        grid=(ne // R,),
        in_specs=[
            pl.BlockSpec((R, H), lambda i: (i, 0)),
            pl.BlockSpec((1, H), lambda i: (0, 0)),
            pl.BlockSpec((H, F), lambda i: (0, 0)),
            pl.BlockSpec((1, F), lambda i: (0, 0)),
        ],
        out_specs=pl.BlockSpec((R, F), lambda i: (i, 0)),
        out_shape=jax.ShapeDtypeStruct((ne, F), jnp.float32),
    )(z, b1, w2, b2)


def _scatter_body(m_hbm, dst_hbm, mself_hbm, out_hbm, acc, di, rows,
                  sem_l, sem_a, *, ne):
    c = lax.axis_index("c")
    s = lax.axis_index("s")

    ept = ne // NS
    nit = ept // SCH
    nsub = SCH // SUB

    for gl in range(2):
        g = 2 * c + gl
        col = g * 32
        pltpu.sync_copy(
            mself_hbm.at[pl.ds(s * NPT, NPT), pl.ds(col, 32)],
            acc.at[pl.ds(s * NPT, NPT), :])
        plsc.subcore_barrier()

        def fire_load(i, b):
            off = s * ept + i * SCH
            pltpu.async_copy(dst_hbm.at[pl.ds(off // SUB, nsub), :],
                             di.at[b], sem_l)
            pltpu.async_copy(m_hbm.at[pl.ds(off, SCH), pl.ds(col, 32)],
                             rows.at[b], sem_l)

        def drain_load(i, b):
            off = s * ept + i * SCH
            pltpu.make_async_copy(dst_hbm.at[pl.ds(off // SUB, nsub), :],
                                  di.at[b], sem_l).wait()
            pltpu.make_async_copy(m_hbm.at[pl.ds(off, SCH), pl.ds(col, 32)],
                                  rows.at[b], sem_l).wait()

        def fire_adds(b):
            for j in range(nsub):
                pltpu.async_copy(rows.at[b, pl.ds(j * SUB, SUB), :],
                                 acc.at[di.at[b, j]], sem_a, add=True)

        def drain_adds(b):
            for j in range(nsub):
                pltpu.make_async_copy(rows.at[b, pl.ds(j * SUB, SUB), :],
                                      acc.at[di.at[b, j]], sem_a).wait()

        fire_load(0, 0)

        def body(i, carry):
            b = lax.rem(i, 2)
            drain_load(i, b)

            @pl.when(i >= 1)
            def _():
                drain_adds(1 - b)

            @pl.when(i + 1 < nit)
            def _():
                fire_load(i + 1, 1 - b)

            fire_adds(b)
            return carry

        lax.fori_loop(0, nit, body, 0)
        drain_adds(lax.rem(nit - 1, 2))
        plsc.subcore_barrier()
        pltpu.sync_copy(
            acc.at[pl.ds(s * NPT, NPT), :],
            out_hbm.at[pl.ds(s * NPT, NPT), pl.ds(col, 32)])
        plsc.subcore_barrier()


def _scatter(m, dst2, mself):
    kernel_fn = functools.partial(
        pl.kernel,
        out_type=jax.ShapeDtypeStruct((N, F), jnp.float32),
        mesh=_sc_mesh(),
        scratch_types=[
            pltpu.VMEM_SHARED((N, 32), jnp.float32),
            pltpu.VMEM((2, SCH // SUB, SUB), jnp.int32),
            pltpu.VMEM((2, SCH, 32), jnp.float32),
            pltpu.SemaphoreType.DMA,
            pltpu.SemaphoreType.DMA,
        ],
        compiler_params=pltpu.CompilerParams(use_tc_tiling_on_sc=False),
    )(functools.partial(_scatter_body, ne=m.shape[0]))
    return kernel_fn(m, dst2, mself)


def _readout_body(y_ref, wr1_ref, br1_ref, wr2_ref, br2_ref, out_ref, acc):
    k = pl.program_id(0)

    @pl.when(k == 0)
    def _():
        acc[...] = jnp.zeros_like(acc)

    acc[...] += jnp.dot(y_ref[...], wr1_ref[...],
                        preferred_element_type=jnp.float32)

    @pl.when(k == pl.num_programs(0) - 1)
    def _():
        h = jnp.maximum(acc[...] + br1_ref[...], 0.0)
        o = jnp.dot(h, wr2_ref[...], preferred_element_type=jnp.float32)
        out_ref[...] = jnp.maximum(o + br2_ref[...], 0.0)


def _readout(y, wr1, br1, wr2, br2):
    K = NPG * F
    KB = 3840
    return pl.pallas_call(
        _readout_body,
        grid=(K // KB,),
        in_specs=[
            pl.BlockSpec((G, KB), lambda i: (0, i)),
            pl.BlockSpec((KB, RO_H), lambda i: (i, 0)),
            pl.BlockSpec((1, RO_H), lambda i: (0, 0)),
            pl.BlockSpec((RO_H, RO_OUT), lambda i: (0, 0)),
            pl.BlockSpec((1, RO_OUT), lambda i: (0, 0)),
        ],
        out_specs=pl.BlockSpec((G, RO_OUT), lambda i: (0, 0)),
        out_shape=jax.ShapeDtypeStruct((G, RO_OUT), jnp.float32),
        scratch_shapes=[pltpu.VMEM((G, RO_H), jnp.float32)],
    )(y, wr1, br1, wr2, br2)


def _layer(h, src2, dst2, w1, b1, w2, b2):
    p, mself = _pre(h, w1, b1, w2, b2)
    eh = E // 2
    rh = eh // SUB
    za = _gather(p, src2, dst2, 0, eh)
    zb = _gather(p, src2, dst2, eh, eh)
    ma = _msg(za, b1, w2, b2)
    mb = _msg(zb, b1, w2, b2)
    xp = _scatter(ma, dst2[:rh], mself)
    return _scatter(mb, dst2[rh:], xp)


def kernel(x, edge_index, W1a, b1a, W2a, b2a, W1b, b1b, W2b, b2b,
           Wr1, br1, Wr2, br2):
    src2 = edge_index[0].reshape(E // SUB, SUB)
    dst2 = edge_index[1].reshape(E // SUB, SUB)
    b1a2 = b1a.reshape(1, H)
    b2a2 = b2a.reshape(1, F)
    b1b2 = b1b.reshape(1, H)
    b2b2 = b2b.reshape(1, F)

    x1 = _layer(x, src2, dst2, W1a, b1a2, W2a, b2a2)
    x2 = _layer(x1, src2, dst2, W1b, b1b2, W2b, b2b2)

    y = x2.reshape(G, NPG * F)
    return _readout(y, Wr1, br1.reshape(1, RO_H), Wr2, br2.reshape(1, RO_OUT))


# speedup vs baseline: 1.8714x; 1.0370x over previous
import functools

import jax
import jax.numpy as jnp
from jax import lax
from jax.experimental import pallas as pl
from jax.experimental.pallas import tpu as pltpu
from jax.experimental.pallas import tpu_sc as plsc

N = 38400
E = 614400
F = 128
H = 16
NPG = 120
G = N // NPG
RO_H = 64
RO_OUT = 960

NC, NS = 2, 16
NW = NC * NS
SUB = 128
CH = 1920
EPW = E // NW
EPT = E // NS
SCH = 768
NPT = N // NS

def _sc_mesh():
    return plsc.VectorSubcoreMesh(core_axis_name="c", subcore_axis_name="s")


def _pre_body(x_ref, w1_ref, b1_ref, w2_ref, b2_ref, p_ref, m_ref):
    p = jnp.dot(x_ref[...], w1_ref[...], preferred_element_type=jnp.float32)
    p_ref[...] = p
    z = jnp.maximum(2.0 * p + b1_ref[...], 0.0)
    m = jnp.dot(z, w2_ref[...], preferred_element_type=jnp.float32) + b2_ref[...]
    m_ref[...] = jnp.maximum(m, 0.0)


def _pre(x, w1, b1, w2, b2):
    R = 4800
    return pl.pallas_call(
        _pre_body,
        grid=(N // R,),
        in_specs=[
            pl.BlockSpec((R, F), lambda i: (i, 0)),
            pl.BlockSpec((F, H), lambda i: (0, 0)),
            pl.BlockSpec((1, H), lambda i: (0, 0)),
            pl.BlockSpec((H, F), lambda i: (0, 0)),
            pl.BlockSpec((1, F), lambda i: (0, 0)),
        ],
        out_specs=[
            pl.BlockSpec((R, H), lambda i: (i, 0)),
            pl.BlockSpec((R, F), lambda i: (i, 0)),
        ],
        out_shape=[
            jax.ShapeDtypeStruct((N, H), jnp.float32),
            jax.ShapeDtypeStruct((N, F), jnp.float32),
        ],
    )(x, w1, b1, w2, b2)


def _gather_body(p_hbm, src_hbm, dst_hbm, z_hbm, ps, si, di, rz,
                 sem_i, sem_g, sem_w, *, e0, ne):
    sc_s = lax.axis_index("s")
    wid = sc_s * NC + lax.axis_index("c")
    epw = ne // NW
    base = e0 + wid * epw
    nit = epw // CH
    nsub = CH // SUB

    pltpu.sync_copy(p_hbm.at[pl.ds(sc_s * NPT, NPT), :],
                    ps.at[pl.ds(sc_s * NPT, NPT), :])
    plsc.subcore_barrier()

    def ids_slab(i):
        return pl.ds((base + i * CH) // SUB, nsub)

    def fire_ids(i, b):
        pltpu.async_copy(src_hbm.at[ids_slab(i), :], si.at[b], sem_i)
        pltpu.async_copy(dst_hbm.at[ids_slab(i), :], di.at[b], sem_i)

    def drain_ids(i, b):
        pltpu.make_async_copy(src_hbm.at[ids_slab(i), :], si.at[b], sem_i).wait()
        pltpu.make_async_copy(dst_hbm.at[ids_slab(i), :], di.at[b], sem_i).wait()

    def fire_rows(i, b, idx, add):
        for j in range(nsub):
            pltpu.async_copy(ps.at[idx.at[b, j]],
                             rz.at[b, pl.ds(j * SUB, SUB), :], sem_g, add=add)

    def drain_rows(i, b, idx):
        for j in range(nsub):
            pltpu.make_async_copy(ps.at[idx.at[b, j]],
                                  rz.at[b, pl.ds(j * SUB, SUB), :], sem_g).wait()

    def fire_wb(i, b):
        pltpu.async_copy(rz.at[b], z_hbm.at[pl.ds(base - e0 + i * CH, CH), :],
                         sem_w)

    def drain_wb(i, b):
        pltpu.make_async_copy(rz.at[b],
                              z_hbm.at[pl.ds(base - e0 + i * CH, CH), :],
                              sem_w).wait()

    fire_ids(0, 0)

    def body(i, carry):
        b = lax.rem(i, 2)
        drain_ids(i, b)
        fire_rows(i, b, si, False)

        @pl.when(i + 1 < nit)
        def _():
            fire_ids(i + 1, 1 - b)

        drain_rows(i, b, si)
        fire_rows(i, b, di, True)

        @pl.when(i >= 1)
        def _():
            drain_wb(i - 1, 1 - b)

        drain_rows(i, b, di)
        fire_wb(i, b)
        return carry

    lax.fori_loop(0, nit, body, 0)
    drain_wb(nit - 1, lax.rem(nit - 1, 2))


def _gather(p, src2, dst2, e0, ne):
    kernel_fn = functools.partial(
        pl.kernel,
        out_type=jax.ShapeDtypeStruct((ne, H), jnp.float32),
        mesh=_sc_mesh(),
        scratch_types=[
            pltpu.VMEM_SHARED((N, H), jnp.float32),
            pltpu.VMEM((2, CH // SUB, SUB), jnp.int32),
            pltpu.VMEM((2, CH // SUB, SUB), jnp.int32),
            pltpu.VMEM((2, CH, H), jnp.float32),
            pltpu.SemaphoreType.DMA,
            pltpu.SemaphoreType.DMA,
            pltpu.SemaphoreType.DMA,
        ],
        compiler_params=pltpu.CompilerParams(use_tc_tiling_on_sc=False),
    )(functools.partial(_gather_body, e0=e0, ne=ne))
    return kernel_fn(p, src2, dst2)


def _msg_body(z_ref, b1_ref, w2_ref, b2_ref, m_ref):
    z = jnp.maximum(z_ref[...] + b1_ref[...], 0.0)
    m = jnp.dot(z, w2_ref[...], preferred_element_type=jnp.float32) + b2_ref[...]
    m_ref[...] = jnp.maximum(m, 0.0)


def _msg(z, b1, w2, b2):
    R = 4096
    ne = z.shape[0]
    return pl.pallas_call(
        _msg_body,
        grid=(ne // R,),
        in_specs=[
            pl.BlockSpec((R, H), lambda i: (i, 0)),
            pl.BlockSpec((1, H), lambda i: (0, 0)),
            pl.BlockSpec((H, F), lambda i: (0, 0)),
            pl.BlockSpec((1, F), lambda i: (0, 0)),
        ],
        out_specs=pl.BlockSpec((R, F), lambda i: (i, 0)),
        out_shape=jax.ShapeDtypeStruct((ne, F), jnp.float32),
    )(z, b1, w2, b2)


def _scatter_body(m_hbm, dst_hbm, mself_hbm, out_hbm, acc, di, rows,
                  sem_l, sem_a, *, ne):
    c = lax.axis_index("c")
    s = lax.axis_index("s")

    ept = ne // NS
    nit = ept // SCH
    nsub = SCH // SUB

    for gl in range(2):
        g = 2 * c + gl
        col = g * 32
        pltpu.sync_copy(
            mself_hbm.at[pl.ds(s * NPT, NPT), pl.ds(col, 32)],
            acc.at[pl.ds(s * NPT, NPT), :])
        plsc.subcore_barrier()

        def fire_load(i, b):
            off = s * ept + i * SCH
            pltpu.async_copy(dst_hbm.at[pl.ds(off // SUB, nsub), :],
                             di.at[b], sem_l)
            pltpu.async_copy(m_hbm.at[pl.ds(off, SCH), pl.ds(col, 32)],
                             rows.at[b], sem_l)

        def drain_load(i, b):
            off = s * ept + i * SCH
            pltpu.make_async_copy(dst_hbm.at[pl.ds(off // SUB, nsub), :],
                                  di.at[b], sem_l).wait()
            pltpu.make_async_copy(m_hbm.at[pl.ds(off, SCH), pl.ds(col, 32)],
                                  rows.at[b], sem_l).wait()

        def fire_adds(b):
            for j in range(nsub):
                pltpu.async_copy(rows.at[b, pl.ds(j * SUB, SUB), :],
                                 acc.at[di.at[b, j]], sem_a, add=True)

        def drain_adds(b):
            for j in range(nsub):
                pltpu.make_async_copy(rows.at[b, pl.ds(j * SUB, SUB), :],
                                      acc.at[di.at[b, j]], sem_a).wait()

        fire_load(0, 0)

        def body(i, carry):
            b = lax.rem(i, 2)
            drain_load(i, b)

            @pl.when(i >= 1)
            def _():
                drain_adds(1 - b)

            @pl.when(i + 1 < nit)
            def _():
                fire_load(i + 1, 1 - b)

            fire_adds(b)
            return carry

        lax.fori_loop(0, nit, body, 0)
        drain_adds(lax.rem(nit - 1, 2))
        plsc.subcore_barrier()
        pltpu.sync_copy(
            acc.at[pl.ds(s * NPT, NPT), :],
            out_hbm.at[pl.ds(s * NPT, NPT), pl.ds(col, 32)])
        plsc.subcore_barrier()


def _scatter(m, dst2, mself):
    kernel_fn = functools.partial(
        pl.kernel,
        out_type=jax.ShapeDtypeStruct((N, F), jnp.float32),
        mesh=_sc_mesh(),
        scratch_types=[
            pltpu.VMEM_SHARED((N, 32), jnp.float32),
            pltpu.VMEM((2, SCH // SUB, SUB), jnp.int32),
            pltpu.VMEM((2, SCH, 32), jnp.float32),
            pltpu.SemaphoreType.DMA,
            pltpu.SemaphoreType.DMA,
        ],
        compiler_params=pltpu.CompilerParams(use_tc_tiling_on_sc=False),
    )(functools.partial(_scatter_body, ne=m.shape[0]))
    return kernel_fn(m, dst2, mself)


def _readout_body(y_ref, wr1_ref, br1_ref, wr2_ref, br2_ref, out_ref, acc):
    k = pl.program_id(0)

    @pl.when(k == 0)
    def _():
        acc[...] = jnp.zeros_like(acc)

    acc[...] += jnp.dot(y_ref[...], wr1_ref[...],
                        preferred_element_type=jnp.float32)

    @pl.when(k == pl.num_programs(0) - 1)
    def _():
        h = jnp.maximum(acc[...] + br1_ref[...], 0.0)
        o = jnp.dot(h, wr2_ref[...], preferred_element_type=jnp.float32)
        out_ref[...] = jnp.maximum(o + br2_ref[...], 0.0)


def _readout(y, wr1, br1, wr2, br2):
    K = NPG * F
    KB = 3840
    return pl.pallas_call(
        _readout_body,
        grid=(K // KB,),
        in_specs=[
            pl.BlockSpec((G, KB), lambda i: (0, i)),
            pl.BlockSpec((KB, RO_H), lambda i: (i, 0)),
            pl.BlockSpec((1, RO_H), lambda i: (0, 0)),
            pl.BlockSpec((RO_H, RO_OUT), lambda i: (0, 0)),
            pl.BlockSpec((1, RO_OUT), lambda i: (0, 0)),
        ],
        out_specs=pl.BlockSpec((G, RO_OUT), lambda i: (0, 0)),
        out_shape=jax.ShapeDtypeStruct((G, RO_OUT), jnp.float32),
        scratch_shapes=[pltpu.VMEM((G, RO_H), jnp.float32)],
    )(y, wr1, br1, wr2, br2)


def _layer(h, src2, dst2, w1, b1, w2, b2):
    p, mself = _pre(h, w1, b1, w2, b2)
    eh = E // 2
    rh = eh // SUB
    za = _gather(p, src2, dst2, 0, eh)
    zb = _gather(p, src2, dst2, eh, eh)
    ma = _msg(za, b1, w2, b2)
    mb = _msg(zb, b1, w2, b2)
    xp = _scatter(ma, dst2[:rh], mself)
    return _scatter(mb, dst2[rh:], xp)


def kernel(x, edge_index, W1a, b1a, W2a, b2a, W1b, b1b, W2b, b2b,
           Wr1, br1, Wr2, br2):
    src2 = edge_index[0].reshape(E // SUB, SUB)
    dst2 = edge_index[1].reshape(E // SUB, SUB)
    b1a2 = b1a.reshape(1, H)
    b2a2 = b2a.reshape(1, F)
    b1b2 = b1b.reshape(1, H)
    b2b2 = b2b.reshape(1, F)

    x1 = _layer(x, src2, dst2, W1a, b1a2, W2a, b2a2)
    x2 = _layer(x1, src2, dst2, W1b, b1b2, W2b, b2b2)

    y = x2.reshape(G, NPG * F)
    return _readout(y, Wr1, br1.reshape(1, RO_H), Wr2, br2.reshape(1, RO_OUT))
